# Initial kernel scaffold; baseline (speedup 1.0000x reference)
import jax
import jax.numpy as jnp
from jax.experimental import pallas as pl


def kernel(x, edge_index, W1, b1, W2, b2):
    raise NotImplementedError("write your pallas kernel here")


# speedup vs baseline: 41.9554x; 41.9554x over previous
import functools

import jax
import jax.numpy as jnp
from jax import lax
from jax.experimental import pallas as pl
from jax.experimental.pallas import tpu as pltpu
from jax.experimental.pallas import tpu_sc as plsc

N = 10000
NROW = 10112
NDEG = 12288
D = 16
DF = 128
NCLS = 40
E = 320000
NBLK = 80
EPW = NBLK * 128
EPAD = 32 * EPW
ROWS_PT = NROW // 16
WIN = NDEG // 32

_mesh = plsc.VectorSubcoreMesh(core_axis_name="c", subcore_axis_name="s")


@functools.partial(
    pl.kernel,
    mesh=_mesh,
    compiler_params=pltpu.CompilerParams(
        needs_layout_passes=False, use_tc_tiling_on_sc=False
    ),
    out_type=jax.ShapeDtypeStruct((NDEG,), jnp.float32),
    scratch_types=[
        pltpu.VMEM((EPW,), jnp.int32),
        pltpu.VMEM((NDEG,), jnp.float32),
        pltpu.VMEM_SHARED((16, NDEG), jnp.float32),
        pltpu.VMEM((16, WIN), jnp.float32),
        pltpu.VMEM((WIN,), jnp.float32),
    ],
)
def _deg_dis(dstF, zdeg, dis_out, dvec, hist, allhist, red, outv):
    c = lax.axis_index("c")
    s = lax.axis_index("s")
    pltpu.sync_copy(zdeg, hist)
    ones = jnp.full((16,), 1.0, dtype=jnp.float32)
    for half in range(2):
        pltpu.sync_copy(dstF.at[2 * s + half], dvec)

        def hloop(q, carry):
            idx16 = dvec[pl.ds(q * 16, 16)]
            plsc.addupdate_scatter(hist, [idx16], ones)
            return carry

        lax.fori_loop(0, EPW // 16, hloop, 0)
    pltpu.sync_copy(hist, allhist.at[s])
    plsc.subcore_barrier()
    w = c * 16 + s

    def rloop(k, carry):
        v = red[0, pl.ds(k * 16, 16)]
        for t in range(1, 16):
            v = v + red[t, pl.ds(k * 16, 16)]
        d = v + 1.0
        i = plsc.bitcast(d, jnp.int32)
        i = 0x5F3759DF - lax.shift_right_logical(i, 1)
        y = plsc.bitcast(i, jnp.float32)
        for _ in range(3):
            y = y * (1.5 - 0.5 * d * y * y)
        outv[pl.ds(k * 16, 16)] = y
        return carry

    pltpu.sync_copy(allhist.at[:, pl.ds(w * WIN, WIN)], red)
    lax.fori_loop(0, WIN // 16, rloop, 0)
    pltpu.sync_copy(outv, dis_out.at[pl.ds(w * WIN, WIN)])


@functools.partial(
    pl.kernel,
    mesh=_mesh,
    compiler_params=pltpu.CompilerParams(use_tc_tiling_on_sc=False),
    out_type=jax.ShapeDtypeStruct((2, NROW, D), jnp.float32),
    scratch_types=[
        pltpu.VMEM((NBLK, 128), jnp.int32),
        pltpu.VMEM((NBLK, 128), jnp.int32),
        pltpu.VMEM((2, 128, D), jnp.float32),
        pltpu.VMEM_SHARED((NROW, D), jnp.float32),
        pltpu.SemaphoreType.DMA,
        pltpu.SemaphoreType.DMA,
    ],
)
def _prop(srcI, dstI, g, zrows, out, sidx, didx, rows, acc, sem0, sem1):
    c = lax.axis_index("c")
    s = lax.axis_index("s")
    wid = s * 2 + c
    pltpu.sync_copy(zrows, acc.at[pl.ds(s * ROWS_PT, ROWS_PT)])
    pltpu.sync_copy(srcI.at[wid], sidx)
    pltpu.sync_copy(dstI.at[wid], didx)
    plsc.subcore_barrier()
    sems = (sem0, sem1)
    for b in range(2):
        pltpu.async_copy(g.at[sidx.at[b]], rows.at[b], sems[b])

    def pair(it, carry):
        i = it * 2
        for b in range(2):
            j = i + b
            pltpu.make_async_copy(g.at[sidx.at[j]], rows.at[b], sems[b]).wait()
            pltpu.sync_copy(rows.at[b], acc.at[didx.at[j]], add=True)

            @pl.when(j + 2 < NBLK)
            def _start():
                pltpu.async_copy(g.at[sidx.at[j + 2]], rows.at[b], sems[b])

        return carry

    lax.fori_loop(0, NBLK // 2, pair, 0)
    plsc.subcore_barrier()
    pltpu.sync_copy(
        acc.at[pl.ds(s * ROWS_PT, ROWS_PT)],
        out.at[c, pl.ds(s * ROWS_PT, ROWS_PT)],
    )


def _mm1_body(x_ref, w_ref, dis_ref, g_ref):
    h = jnp.dot(x_ref[...], w_ref[...], preferred_element_type=jnp.float32)
    g_ref[...] = h * dis_ref[...]


def _mid_body(p_ref, g_ref, dis_ref, b_ref, o_ref):
    su = (p_ref[0] + p_ref[1] + g_ref[...]) * dis_ref[...]
    h2 = jnp.maximum(su + b_ref[...], 0.0)
    o_ref[...] = h2 * dis_ref[...]


def _final_body(p_ref, g_ref, dis_ref, w_ref, b_ref, o_ref):
    ah = (p_ref[0] + p_ref[1] + g_ref[...]) * dis_ref[...]
    o_ref[...] = (
        jnp.dot(ah, w_ref[...], preferred_element_type=jnp.float32) + b_ref[...]
    )


_mm1 = pl.pallas_call(
    _mm1_body, out_shape=jax.ShapeDtypeStruct((NROW, D), jnp.float32)
)
_mid = pl.pallas_call(
    _mid_body, out_shape=jax.ShapeDtypeStruct((NROW, D), jnp.float32)
)
_final = pl.pallas_call(
    _final_body, out_shape=jax.ShapeDtypeStruct((NROW, NCLS), jnp.float32)
)


def kernel(x, edge_index, W1, b1, W2, b2):
    src = edge_index[0].astype(jnp.int32)
    dst = edge_index[1].astype(jnp.int32)
    npad = EPAD - E
    srcp = jnp.concatenate([src, jnp.zeros((npad,), jnp.int32)])
    dstp = jnp.concatenate(
        [dst, N + (jnp.arange(npad, dtype=jnp.int32) % 16)]
    )
    srcI = srcp.reshape(32, NBLK, 128)
    dstI = dstp.reshape(32, NBLK, 128)
    dstF = dstp.reshape(32, EPW)
    zdeg = jnp.zeros((NDEG,), jnp.float32)
    zrows = jnp.zeros((ROWS_PT, D), jnp.float32)
    xp = jnp.concatenate([x, jnp.zeros((NROW - N, DF), x.dtype)])

    dis = _deg_dis(dstF, zdeg)
    dis_col = dis[:NROW].reshape(NROW, 1)
    g1 = _mm1(xp, W1, dis_col)
    p1 = _prop(srcI, dstI, g1, zrows)
    g2 = _mid(p1, g1, dis_col, b1.reshape(1, D))
    p2 = _prop(srcI, dstI, g2, zrows)
    out = _final(p2, g2, dis_col, W2, b2.reshape(1, NCLS))
    return out[:N]


# speedup vs baseline: 43.6827x; 1.0412x over previous
import functools

import jax
import jax.numpy as jnp
from jax import lax
from jax.experimental import pallas as pl
from jax.experimental.pallas import tpu as pltpu
from jax.experimental.pallas import tpu_sc as plsc

N = 10000
NROW = 10112
NDEG = 12288
D = 16
DF = 128
NCLS = 40
E = 320000
NBLK = 80
EPW = NBLK * 128
EPAD = 32 * EPW
ROWS_PT = NROW // 16
WIN = NDEG // 32

_mesh = plsc.VectorSubcoreMesh(core_axis_name="c", subcore_axis_name="s")


@functools.partial(
    pl.kernel,
    mesh=_mesh,
    compiler_params=pltpu.CompilerParams(
        needs_layout_passes=False, use_tc_tiling_on_sc=False
    ),
    out_type=jax.ShapeDtypeStruct((NDEG,), jnp.float32),
    scratch_types=[
        pltpu.VMEM((EPW,), jnp.int32),
        pltpu.VMEM((NDEG,), jnp.float32),
        pltpu.VMEM_SHARED((16, NDEG), jnp.float32),
        pltpu.VMEM((16, WIN), jnp.float32),
        pltpu.VMEM((WIN,), jnp.float32),
    ],
)
def _deg_dis(dstF, zdeg, dis_out, dvec, hist, allhist, red, outv):
    c = lax.axis_index("c")
    s = lax.axis_index("s")
    pltpu.sync_copy(zdeg, hist)
    ones = jnp.full((16,), 1.0, dtype=jnp.float32)
    for half in range(2):
        pltpu.sync_copy(dstF.at[2 * s + half], dvec)

        def hloop(q, carry):
            idx16 = dvec[pl.ds(q * 16, 16)]
            plsc.addupdate_scatter(hist, [idx16], ones)
            return carry

        lax.fori_loop(0, EPW // 16, hloop, 0)
    pltpu.sync_copy(hist, allhist.at[s])
    plsc.subcore_barrier()
    w = c * 16 + s

    def rloop(k, carry):
        v = red[0, pl.ds(k * 16, 16)]
        for t in range(1, 16):
            v = v + red[t, pl.ds(k * 16, 16)]
        d = v + 1.0
        i = plsc.bitcast(d, jnp.int32)
        i = 0x5F3759DF - lax.shift_right_logical(i, 1)
        y = plsc.bitcast(i, jnp.float32)
        for _ in range(3):
            y = y * (1.5 - 0.5 * d * y * y)
        outv[pl.ds(k * 16, 16)] = y
        return carry

    pltpu.sync_copy(allhist.at[:, pl.ds(w * WIN, WIN)], red)
    lax.fori_loop(0, WIN // 16, rloop, 0)
    pltpu.sync_copy(outv, dis_out.at[pl.ds(w * WIN, WIN)])


@functools.partial(
    pl.kernel,
    mesh=_mesh,
    compiler_params=pltpu.CompilerParams(use_tc_tiling_on_sc=False),
    out_type=jax.ShapeDtypeStruct((2, NROW, D), jnp.float32),
    scratch_types=[
        pltpu.VMEM((NBLK, 128), jnp.int32),
        pltpu.VMEM((NBLK, 128), jnp.int32),
        pltpu.VMEM((8, 128, D), jnp.float32),
        pltpu.VMEM_SHARED((NROW, D), jnp.float32),
        [pltpu.SemaphoreType.DMA] * 8,
        [pltpu.SemaphoreType.DMA] * 8,
    ],
)
def _prop(srcI, dstI, g, zrows, out, sidx, didx, rows, acc, gsem, ssem):
    c = lax.axis_index("c")
    s = lax.axis_index("s")
    wid = s * 2 + c
    pltpu.sync_copy(zrows, acc.at[pl.ds(s * ROWS_PT, ROWS_PT)])
    pltpu.sync_copy(srcI.at[wid], sidx)
    pltpu.sync_copy(dstI.at[wid], didx)
    plsc.subcore_barrier()
    for b in range(4):
        pltpu.async_copy(g.at[sidx.at[b]], rows.at[b], gsem[b])

    def octet(it, carry):
        i = it * 8
        for b in range(8):
            j = i + b
            pltpu.make_async_copy(g.at[sidx.at[j]], rows.at[b], gsem[b]).wait()
            pltpu.async_copy(rows.at[b], acc.at[didx.at[j]], ssem[b], add=True)
            j4 = j + 4
            b4 = (b + 4) % 8

            @pl.when(j4 < NBLK)
            def _ahead():
                @pl.when(j4 >= 8)
                def _drain():
                    pltpu.make_async_copy(
                        rows.at[b4], acc.at[didx.at[j4 - 8]], ssem[b4]
                    ).wait()

                pltpu.async_copy(g.at[sidx.at[j4]], rows.at[b4], gsem[b4])

        return carry

    lax.fori_loop(0, NBLK // 8, octet, 0)
    for b in range(8):
        pltpu.make_async_copy(
            rows.at[b], acc.at[didx.at[NBLK - 8 + b]], ssem[b]
        ).wait()
    plsc.subcore_barrier()
    pltpu.sync_copy(
        acc.at[pl.ds(s * ROWS_PT, ROWS_PT)],
        out.at[c, pl.ds(s * ROWS_PT, ROWS_PT)],
    )


def _mm1_body(x_ref, w_ref, dis_ref, g_ref):
    h = jnp.dot(x_ref[...], w_ref[...], preferred_element_type=jnp.float32)
    g_ref[...] = h * dis_ref[...]


def _mid_body(p_ref, g_ref, dis_ref, b_ref, o_ref):
    su = (p_ref[0] + p_ref[1] + g_ref[...]) * dis_ref[...]
    h2 = jnp.maximum(su + b_ref[...], 0.0)
    o_ref[...] = h2 * dis_ref[...]


def _final_body(p_ref, g_ref, dis_ref, w_ref, b_ref, o_ref):
    ah = (p_ref[0] + p_ref[1] + g_ref[...]) * dis_ref[...]
    o_ref[...] = (
        jnp.dot(ah, w_ref[...], preferred_element_type=jnp.float32) + b_ref[...]
    )


_mm1 = pl.pallas_call(
    _mm1_body, out_shape=jax.ShapeDtypeStruct((NROW, D), jnp.float32)
)
_mid = pl.pallas_call(
    _mid_body, out_shape=jax.ShapeDtypeStruct((NROW, D), jnp.float32)
)
_final = pl.pallas_call(
    _final_body, out_shape=jax.ShapeDtypeStruct((NROW, NCLS), jnp.float32)
)


def kernel(x, edge_index, W1, b1, W2, b2):
    src = edge_index[0].astype(jnp.int32)
    dst = edge_index[1].astype(jnp.int32)
    npad = EPAD - E
    srcp = jnp.concatenate([src, jnp.zeros((npad,), jnp.int32)])
    dstp = jnp.concatenate(
        [dst, N + (jnp.arange(npad, dtype=jnp.int32) % 16)]
    )
    srcI = srcp.reshape(32, NBLK, 128)
    dstI = dstp.reshape(32, NBLK, 128)
    dstF = dstp.reshape(32, EPW)
    zdeg = jnp.zeros((NDEG,), jnp.float32)
    zrows = jnp.zeros((ROWS_PT, D), jnp.float32)
    xp = jnp.concatenate([x, jnp.zeros((NROW - N, DF), x.dtype)])

    dis = _deg_dis(dstF, zdeg)
    dis_col = dis[:NROW].reshape(NROW, 1)
    g1 = _mm1(xp, W1, dis_col)
    p1 = _prop(srcI, dstI, g1, zrows)
    g2 = _mid(p1, g1, dis_col, b1.reshape(1, D))
    p2 = _prop(srcI, dstI, g2, zrows)
    out = _final(p2, g2, dis_col, W2, b2.reshape(1, NCLS))
    return out[:N]


# speedup vs baseline: 62.3693x; 1.4278x over previous
import functools

import jax
import jax.numpy as jnp
from jax import lax
from jax.experimental import pallas as pl
from jax.experimental.pallas import tpu as pltpu
from jax.experimental.pallas import tpu_sc as plsc

N = 10000
NROW = 10112
NDEG = 12288
D = 16
DF = 128
NCLS = 40
E = 320000
NBLK = 80
EPW = NBLK * 128
EPAD = 32 * EPW
ROWS_PT = NROW // 16
WIN = NDEG // 32

_mesh = plsc.VectorSubcoreMesh(core_axis_name="c", subcore_axis_name="s")


@functools.partial(
    pl.kernel,
    mesh=_mesh,
    compiler_params=pltpu.CompilerParams(
        needs_layout_passes=False, use_tc_tiling_on_sc=False
    ),
    out_type=jax.ShapeDtypeStruct((NDEG,), jnp.float32),
    scratch_types=[
        pltpu.VMEM((NBLK, 128), jnp.int32),
        pltpu.VMEM((NDEG,), jnp.float32),
        pltpu.VMEM_SHARED((16, NDEG), jnp.float32),
        pltpu.VMEM((16, WIN), jnp.float32),
        pltpu.VMEM((WIN,), jnp.float32),
    ],
)
def _deg_dis(eIp, zdeg, dis_out, dvec, hist, allhist, red, outv):
    c = lax.axis_index("c")
    s = lax.axis_index("s")
    pltpu.sync_copy(zdeg, hist)
    ones = jnp.full((16,), 1.0, dtype=jnp.float32)
    for half in range(2):
        pltpu.sync_copy(eIp.at[1, 2 * s + half], dvec)

        def hloop(j, carry):
            for k in range(8):
                idx16 = dvec[j, pl.ds(k * 16, 16)]
                plsc.addupdate_scatter(hist, [idx16], ones)
            return carry

        lax.fori_loop(0, NBLK, hloop, 0)
    pltpu.sync_copy(hist, allhist.at[s])
    plsc.subcore_barrier()
    w = c * 16 + s

    def rloop(k, carry):
        v = red[0, pl.ds(k * 16, 16)]
        for t in range(1, 16):
            v = v + red[t, pl.ds(k * 16, 16)]
        d = v + 1.0
        i = plsc.bitcast(d, jnp.int32)
        i = 0x5F3759DF - lax.shift_right_logical(i, 1)
        y = plsc.bitcast(i, jnp.float32)
        for _ in range(3):
            y = y * (1.5 - 0.5 * d * y * y)
        outv[pl.ds(k * 16, 16)] = y
        return carry

    pltpu.sync_copy(allhist.at[:, pl.ds(w * WIN, WIN)], red)
    lax.fori_loop(0, WIN // 16, rloop, 0)
    pltpu.sync_copy(outv, dis_out.at[pl.ds(w * WIN, WIN)])


@functools.partial(
    pl.kernel,
    mesh=_mesh,
    compiler_params=pltpu.CompilerParams(use_tc_tiling_on_sc=False),
    out_type=jax.ShapeDtypeStruct((2, NROW, D), jnp.float32),
    scratch_types=[
        pltpu.VMEM((NBLK, 128), jnp.int32),
        pltpu.VMEM((NBLK, 128), jnp.int32),
        pltpu.VMEM((8, 128, D), jnp.float32),
        pltpu.VMEM_SHARED((NROW, D), jnp.float32),
        pltpu.VMEM_SHARED((NROW, D), jnp.float32),
        [pltpu.SemaphoreType.DMA] * 8,
        [pltpu.SemaphoreType.DMA] * 8,
    ],
)
def _prop(eIp, g, zrows, out, sidx, didx, rows, acc, gs, gsem, ssem):
    c = lax.axis_index("c")
    s = lax.axis_index("s")
    wid = s * 2 + c
    pltpu.sync_copy(zrows, acc.at[pl.ds(s * ROWS_PT, ROWS_PT)])
    pltpu.sync_copy(
        g.at[pl.ds(s * ROWS_PT, ROWS_PT)], gs.at[pl.ds(s * ROWS_PT, ROWS_PT)]
    )
    pltpu.sync_copy(eIp.at[0, wid], sidx)
    pltpu.sync_copy(eIp.at[1, wid], didx)
    plsc.subcore_barrier()
    for b in range(4):
        pltpu.async_copy(gs.at[sidx.at[b]], rows.at[b], gsem[b])

    def octet(it, carry):
        i = it * 8
        for b in range(8):
            j = i + b
            pltpu.make_async_copy(gs.at[sidx.at[j]], rows.at[b], gsem[b]).wait()
            pltpu.async_copy(rows.at[b], acc.at[didx.at[j]], ssem[b], add=True)
            j4 = j + 4
            b4 = (b + 4) % 8

            @pl.when(j4 < NBLK)
            def _ahead():
                @pl.when(j4 >= 8)
                def _drain():
                    pltpu.make_async_copy(
                        rows.at[b4], acc.at[didx.at[j4 - 8]], ssem[b4]
                    ).wait()

                pltpu.async_copy(gs.at[sidx.at[j4]], rows.at[b4], gsem[b4])

        return carry

    lax.fori_loop(0, NBLK // 8, octet, 0)
    for b in range(8):
        pltpu.make_async_copy(
            rows.at[b], acc.at[didx.at[NBLK - 8 + b]], ssem[b]
        ).wait()
    plsc.subcore_barrier()
    pltpu.sync_copy(
        acc.at[pl.ds(s * ROWS_PT, ROWS_PT)],
        out.at[c, pl.ds(s * ROWS_PT, ROWS_PT)],
    )


def _mm1_body(x_ref, w_ref, dis_ref, g_ref):
    h = jnp.dot(x_ref[...], w_ref[...], preferred_element_type=jnp.float32)
    g_ref[pl.ds(0, N), :] = h * dis_ref[...]


def _mid_body(p_ref, g_ref, dis_ref, b_ref, o_ref):
    su = (p_ref[0] + p_ref[1] + g_ref[...]) * dis_ref[...]
    h2 = jnp.maximum(su + b_ref[...], 0.0)
    o_ref[...] = h2 * dis_ref[...]


def _final_body(p_ref, g_ref, dis_ref, w_ref, b_ref, o_ref):
    ah = (p_ref[0] + p_ref[1] + g_ref[...]) * dis_ref[...]
    o_ref[...] = (
        jnp.dot(ah, w_ref[...], preferred_element_type=jnp.float32) + b_ref[...]
    )


_mm1 = pl.pallas_call(
    _mm1_body, out_shape=jax.ShapeDtypeStruct((NROW, D), jnp.float32)
)
_mid = pl.pallas_call(
    _mid_body, out_shape=jax.ShapeDtypeStruct((NROW, D), jnp.float32)
)
_final = pl.pallas_call(
    _final_body, out_shape=jax.ShapeDtypeStruct((NROW, NCLS), jnp.float32)
)


def kernel(x, edge_index, W1, b1, W2, b2):
    npad = EPAD - E
    padblk = jnp.stack(
        [
            jnp.zeros((npad,), jnp.int32),
            N + (jnp.arange(npad, dtype=jnp.int32) % 16),
        ]
    )
    eIp = jnp.concatenate(
        [edge_index.astype(jnp.int32), padblk], axis=1
    ).reshape(2, 32, NBLK, 128)
    zdeg = jnp.zeros((NDEG,), jnp.float32)
    zrows = jnp.zeros((ROWS_PT, D), jnp.float32)

    dis = _deg_dis(eIp, zdeg)
    dis_colN = dis[:N].reshape(N, 1)
    dis_colR = dis[:NROW].reshape(NROW, 1)
    g1 = _mm1(x, W1, dis_colN)
    p1 = _prop(eIp, g1, zrows)
    g2 = _mid(p1, g1, dis_colR, b1.reshape(1, D))
    p2 = _prop(eIp, g2, zrows)
    out = _final(p2, g2, dis_colR, W2, b2.reshape(1, NCLS))
    return out[:N]


# speedup vs baseline: 84.0275x; 1.3473x over previous
import functools

import jax
import jax.numpy as jnp
from jax import lax
from jax.experimental import pallas as pl
from jax.experimental.pallas import tpu as pltpu
from jax.experimental.pallas import tpu_sc as plsc

N = 10000
NROW = 10112
FL = NROW // 8
NDEG = 12288
D = 16
DF = 128
NCLS = 40
E = 320000
NBLK = 80
EPW = NBLK * 128
EPAD = 32 * EPW
ROWS_PT = NROW // 16
FL_PT = FL // 16
WIN = NDEG // 32

_mesh = plsc.VectorSubcoreMesh(core_axis_name="c", subcore_axis_name="s")


@functools.partial(
    pl.kernel,
    mesh=_mesh,
    compiler_params=pltpu.CompilerParams(
        needs_layout_passes=False, use_tc_tiling_on_sc=False
    ),
    out_type=jax.ShapeDtypeStruct((NDEG * 16,), jnp.float32),
    scratch_types=[
        pltpu.VMEM((NBLK, 128), jnp.int32),
        pltpu.VMEM((NDEG,), jnp.float32),
        pltpu.VMEM_SHARED((16, NDEG), jnp.float32),
        pltpu.VMEM((16, WIN), jnp.float32),
        pltpu.VMEM((WIN * 16,), jnp.float32),
    ],
)
def _deg_dis(eIp, zdeg, dis_out, dvec, hist, allhist, red, outv):
    c = lax.axis_index("c")
    s = lax.axis_index("s")
    pltpu.sync_copy(zdeg, hist)
    ones = jnp.full((16,), 1.0, dtype=jnp.float32)
    for half in range(2):
        pltpu.sync_copy(eIp.at[1, 2 * s + half], dvec)

        def hloop(j, carry):
            for k in range(8):
                idx16 = dvec[j, pl.ds(k * 16, 16)]
                plsc.addupdate_scatter(hist, [idx16], ones)
            return carry

        lax.fori_loop(0, NBLK, hloop, 0)
    pltpu.sync_copy(hist, allhist.at[s])
    plsc.subcore_barrier()
    w = c * 16 + s

    def rloop(k, carry):
        v = red[0, pl.ds(k * 16, 16)]
        for t in range(1, 16):
            v = v + red[t, pl.ds(k * 16, 16)]
        d = v + 1.0
        i = plsc.bitcast(d, jnp.int32)
        i = 0x5F3759DF - lax.shift_right_logical(i, 1)
        y = plsc.bitcast(i, jnp.float32)
        for _ in range(3):
            y = y * (1.5 - 0.5 * d * y * y)
        for i16 in range(16):
            outv[pl.ds((k * 16 + i16) * 16, 16)] = jnp.broadcast_to(
                y[i16], (16,)
            )
        return carry

    pltpu.sync_copy(allhist.at[:, pl.ds(w * WIN, WIN)], red)
    lax.fori_loop(0, WIN // 16, rloop, 0)
    pltpu.sync_copy(outv, dis_out.at[pl.ds(w * WIN * 16, WIN * 16)])


@functools.partial(
    pl.kernel,
    mesh=_mesh,
    compiler_params=pltpu.CompilerParams(use_tc_tiling_on_sc=False),
    out_type=jax.ShapeDtypeStruct((2, NROW, D), jnp.float32),
    scratch_types=[
        pltpu.VMEM((NBLK, 128), jnp.int32),
        pltpu.VMEM((NBLK, 128), jnp.int32),
        pltpu.VMEM((8, 128, D), jnp.float32),
        pltpu.VMEM((FL_PT, 128), jnp.float32),
        pltpu.VMEM((FL_PT, 128), jnp.float32),
        pltpu.VMEM((ROWS_PT, D), jnp.float32),
        pltpu.VMEM_SHARED((NROW, D), jnp.float32),
        pltpu.VMEM_SHARED((NROW, D), jnp.float32),
        [pltpu.SemaphoreType.DMA] * 8,
        [pltpu.SemaphoreType.DMA] * 8,
    ],
)
def _prop(eIp, hf, d16, zrows, out, sidx, didx, rows, hbuf, dbuf, sbuf,
          acc, gs, gsem, ssem):
    c = lax.axis_index("c")
    s = lax.axis_index("s")
    wid = s * 2 + c
    pltpu.sync_copy(zrows, acc.at[pl.ds(s * ROWS_PT, ROWS_PT)])
    pltpu.sync_copy(hf.at[pl.ds(s * FL_PT, FL_PT)], hbuf)
    pltpu.sync_copy(d16.at[pl.ds(s * FL_PT, FL_PT)], dbuf)

    def scale(j, carry):
        for k in range(8):
            sbuf[j * 8 + k, :] = (
                hbuf[j, pl.ds(k * 16, 16)] * dbuf[j, pl.ds(k * 16, 16)]
            )
        return carry

    lax.fori_loop(0, FL_PT, scale, 0)
    pltpu.sync_copy(sbuf, gs.at[pl.ds(s * ROWS_PT, ROWS_PT)])
    pltpu.sync_copy(eIp.at[0, wid], sidx)
    pltpu.sync_copy(eIp.at[1, wid], didx)
    plsc.subcore_barrier()
    for b in range(4):
        pltpu.async_copy(gs.at[sidx.at[b]], rows.at[b], gsem[b])

    def octet(it, carry):
        i = it * 8
        for b in range(8):
            j = i + b
            pltpu.make_async_copy(gs.at[sidx.at[j]], rows.at[b], gsem[b]).wait()
            pltpu.async_copy(rows.at[b], acc.at[didx.at[j]], ssem[b], add=True)
            j4 = j + 4
            b4 = (b + 4) % 8

            @pl.when(j4 < NBLK)
            def _ahead():
                @pl.when(j4 >= 8)
                def _drain():
                    pltpu.make_async_copy(
                        rows.at[b4], acc.at[didx.at[j4 - 8]], ssem[b4]
                    ).wait()

                pltpu.async_copy(gs.at[sidx.at[j4]], rows.at[b4], gsem[b4])

        return carry

    lax.fori_loop(0, NBLK // 8, octet, 0)
    for b in range(8):
        pltpu.make_async_copy(
            rows.at[b], acc.at[didx.at[NBLK - 8 + b]], ssem[b]
        ).wait()
    plsc.subcore_barrier()
    pltpu.sync_copy(
        acc.at[pl.ds(s * ROWS_PT, ROWS_PT)],
        out.at[c, pl.ds(s * ROWS_PT, ROWS_PT)],
    )


def _mm1_body(x_ref, w_ref, o_ref):
    h = jnp.dot(x_ref[...], w_ref[...], preferred_element_type=jnp.float32)
    o_ref[pl.ds(0, N), :] = h


def _mid_body(p_ref, h_ref, d_ref, b_ref, o_ref):
    d16 = d_ref[...]
    su = d16 * (p_ref[0] + p_ref[1] + d16 * h_ref[...])
    o_ref[...] = jnp.maximum(su + b_ref[...], 0.0)


def _final_body(p_ref, h_ref, d_ref, w_ref, b_ref, o_ref):
    d16 = d_ref[...]
    ah = d16 * (p_ref[0] + p_ref[1] + d16 * h_ref[...])
    o_ref[...] = (
        jnp.dot(ah, w_ref[...], preferred_element_type=jnp.float32) + b_ref[...]
    )


_mm1 = pl.pallas_call(
    _mm1_body, out_shape=jax.ShapeDtypeStruct((NROW, D), jnp.float32)
)
_mid = pl.pallas_call(
    _mid_body, out_shape=jax.ShapeDtypeStruct((FL, 128), jnp.float32)
)
_final = pl.pallas_call(
    _final_body, out_shape=jax.ShapeDtypeStruct((FL, 8 * NCLS), jnp.float32)
)


def kernel(x, edge_index, W1, b1, W2, b2):
    npad = EPAD - E
    padblk = jnp.stack(
        [
            jnp.zeros((npad,), jnp.int32),
            N + (jnp.arange(npad, dtype=jnp.int32) % 16),
        ]
    )
    eIp = jnp.concatenate(
        [edge_index.astype(jnp.int32), padblk], axis=1
    ).reshape(2, 32, NBLK, 128)
    zdeg = jnp.zeros((NDEG,), jnp.float32)
    zrows = jnp.zeros((ROWS_PT, D), jnp.float32)
    b1f = jnp.tile(b1, 8).reshape(1, 128)
    w2b = jax.scipy.linalg.block_diag(*([W2] * 8))
    b2t = jnp.tile(b2, 8).reshape(1, 8 * NCLS)

    dis16 = _deg_dis(eIp, zdeg)[: NROW * 16].reshape(FL, 128)
    h1 = _mm1(x, W1).reshape(FL, 128)
    p1 = _prop(eIp, h1, dis16, zrows).reshape(2, FL, 128)
    h2 = _mid(p1, h1, dis16, b1f)
    p2 = _prop(eIp, h2, dis16, zrows).reshape(2, FL, 128)
    out = _final(p2, h2, dis16, w2b, b2t)
    return out.reshape(NROW, NCLS)[:N]
